# Initial kernel scaffold; baseline (speedup 1.0000x reference)
import jax
import jax.numpy as jnp
from jax.experimental import pallas as pl


def kernel(h, label, W1, b1, Wa, ba, Wb, bb, Wc, bc, Wcls, bcls):
    raise NotImplementedError("write your pallas kernel here")


# speedup vs baseline: 1.1395x; 1.1395x over previous
import functools

import jax
import jax.numpy as jnp
from jax.experimental import pallas as pl
from jax.experimental.pallas import tpu as pltpu

N, L, D1, D2, C = 50000, 1024, 256, 128, 2
TILE = 1000
NUM_TILES = N // TILE


def _clam_kernel(h_ref, w1_ref, b1_ref, wa_ref, ba_ref, wb_ref, bb_ref,
                 wc_ref, wcls_ref, bcls_ref,
                 logits_ref, prob_ref, yhat_ref,
                 acc_ref, m_ref, s_ref):
    i = pl.program_id(0)

    @pl.when(i == 0)
    def _init():
        acc_ref[...] = jnp.zeros_like(acc_ref)
        m_ref[...] = jnp.full((1, 1), -jnp.inf, jnp.float32)
        s_ref[...] = jnp.zeros((1, 1), jnp.float32)

    hh = jax.nn.relu(
        jnp.dot(h_ref[...], w1_ref[...], preferred_element_type=jnp.float32)
        + b1_ref[...])
    a = jnp.tanh(
        jnp.dot(hh, wa_ref[...], preferred_element_type=jnp.float32)
        + ba_ref[...])
    g = jax.nn.sigmoid(
        jnp.dot(hh, wb_ref[...], preferred_element_type=jnp.float32)
        + bb_ref[...])
    sc = jnp.dot(a * g, wc_ref[...],
                 preferred_element_type=jnp.float32)

    m_old = m_ref[...]
    m_new = jnp.maximum(m_old, jnp.max(sc, axis=0, keepdims=True))
    alpha = jnp.exp(m_old - m_new)
    p = jnp.exp(sc - m_new)
    s_ref[...] = s_ref[...] * alpha + jnp.sum(p, axis=0, keepdims=True)
    acc_ref[...] = acc_ref[...] * alpha + jnp.dot(
        p.T, hh, preferred_element_type=jnp.float32)
    m_ref[...] = m_new

    @pl.when(i == NUM_TILES - 1)
    def _finish():
        m_vec = acc_ref[...] / s_ref[...]
        logits = jnp.dot(m_vec, wcls_ref[...],
                         preferred_element_type=jnp.float32) + bcls_ref[...]
        logits_ref[...] = logits
        z = logits - jnp.max(logits, axis=1, keepdims=True)
        ez = jnp.exp(z)
        prob_ref[...] = ez / jnp.sum(ez, axis=1, keepdims=True)
        yhat_ref[...] = (logits[:, 1:2] > logits[:, 0:1]).astype(jnp.int32)


@functools.partial(jax.jit, static_argnames=())
def _run(h, W1, b1, Wa, ba, Wb, bb, Wc, Wcls, bcls):
    out_shapes = (
        jax.ShapeDtypeStruct((1, C), jnp.float32),
        jax.ShapeDtypeStruct((1, C), jnp.float32),
        jax.ShapeDtypeStruct((1, 1), jnp.int32),
    )
    full = lambda shape: pl.BlockSpec(shape, lambda i: (0, 0))
    return pl.pallas_call(
        _clam_kernel,
        grid=(NUM_TILES,),
        in_specs=[
            pl.BlockSpec((TILE, L), lambda i: (i, 0)),
            full((L, D1)),
            full((1, D1)),
            full((D1, D2)),
            full((1, D2)),
            full((D1, D2)),
            full((1, D2)),
            full((D2, 1)),
            full((D1, C)),
            full((1, C)),
        ],
        out_specs=(full((1, C)), full((1, C)), full((1, 1))),
        out_shape=out_shapes,
        scratch_shapes=[
            pltpu.VMEM((1, D1), jnp.float32),
            pltpu.VMEM((1, 1), jnp.float32),
            pltpu.VMEM((1, 1), jnp.float32),
        ],
    )(h, W1, b1, Wa, ba, Wb, bb, Wc, Wcls, bcls)


def kernel(h, label, W1, b1, Wa, ba, Wb, bb, Wc, bc, Wcls, bcls):
    del label, bc
    logits, prob, yhat = _run(
        h, W1, b1.reshape(1, D1), Wa, ba.reshape(1, D2),
        Wb, bb.reshape(1, D2), Wc, Wcls, bcls.reshape(1, C))
    return (logits, prob, yhat)


# speedup vs baseline: 1.4447x; 1.2678x over previous
import functools

import jax
import jax.numpy as jnp
from jax.experimental import pallas as pl
from jax.experimental.pallas import tpu as pltpu

N, L, D1, D2, C = 50000, 1024, 256, 128, 2
TILE = 2000
NUM_TILES = N // TILE


def _clam_kernel(h_ref, w1_ref, b1_ref, wab_ref, bab_ref,
                 wc_ref, wcls_ref, bcls_ref,
                 logits_ref, prob_ref, yhat_ref,
                 acc_ref, m_ref, s_ref):
    i = pl.program_id(0)

    @pl.when(i == 0)
    def _init():
        acc_ref[...] = jnp.zeros_like(acc_ref)
        m_ref[...] = jnp.full((1, 1), -jnp.inf, jnp.float32)
        s_ref[...] = jnp.zeros((1, 1), jnp.float32)

    hh = jax.nn.relu(
        jnp.dot(h_ref[...].astype(jnp.bfloat16), w1_ref[...],
                preferred_element_type=jnp.float32)
        + b1_ref[...])
    ag = jnp.dot(hh.astype(jnp.bfloat16), wab_ref[...],
                 preferred_element_type=jnp.float32) + bab_ref[...]
    a = jnp.tanh(ag[:, :D2])
    g = jax.nn.sigmoid(ag[:, D2:])
    sc = jnp.dot(a * g, wc_ref[...],
                 preferred_element_type=jnp.float32)

    m_old = m_ref[...]
    m_new = jnp.maximum(m_old, jnp.max(sc, axis=0, keepdims=True))
    alpha = jnp.exp(m_old - m_new)
    p = jnp.exp(sc - m_new)
    s_ref[...] = s_ref[...] * alpha + jnp.sum(p, axis=0, keepdims=True)
    acc_ref[...] = acc_ref[...] * alpha + jnp.dot(
        p.T, hh, preferred_element_type=jnp.float32)
    m_ref[...] = m_new

    @pl.when(i == NUM_TILES - 1)
    def _finish():
        m_vec = acc_ref[...] / s_ref[...]
        logits = jnp.dot(m_vec, wcls_ref[...],
                         preferred_element_type=jnp.float32) + bcls_ref[...]
        logits_ref[...] = logits
        z = logits - jnp.max(logits, axis=1, keepdims=True)
        ez = jnp.exp(z)
        prob_ref[...] = ez / jnp.sum(ez, axis=1, keepdims=True)
        yhat_ref[...] = (logits[:, 1:2] > logits[:, 0:1]).astype(jnp.int32)


@jax.jit
def _run(h, W1, b1, Wab, bab, Wc, Wcls, bcls):
    out_shapes = (
        jax.ShapeDtypeStruct((1, C), jnp.float32),
        jax.ShapeDtypeStruct((1, C), jnp.float32),
        jax.ShapeDtypeStruct((1, 1), jnp.int32),
    )
    full = lambda shape: pl.BlockSpec(shape, lambda i: (0, 0))
    return pl.pallas_call(
        _clam_kernel,
        grid=(NUM_TILES,),
        in_specs=[
            pl.BlockSpec((TILE, L), lambda i: (i, 0)),
            full((L, D1)),
            full((1, D1)),
            full((D1, 2 * D2)),
            full((1, 2 * D2)),
            full((D2, 1)),
            full((D1, C)),
            full((1, C)),
        ],
        out_specs=(full((1, C)), full((1, C)), full((1, 1))),
        out_shape=out_shapes,
        scratch_shapes=[
            pltpu.VMEM((1, D1), jnp.float32),
            pltpu.VMEM((1, 1), jnp.float32),
            pltpu.VMEM((1, 1), jnp.float32),
        ],
    )(h, W1, b1, Wab, bab, Wc, Wcls, bcls)


def kernel(h, label, W1, b1, Wa, ba, Wb, bb, Wc, bc, Wcls, bcls):
    del label, bc
    Wab = jnp.concatenate([Wa, Wb], axis=1).astype(jnp.bfloat16)
    bab = jnp.concatenate([ba, bb]).reshape(1, 2 * D2)
    logits, prob, yhat = _run(
        h, W1.astype(jnp.bfloat16), b1.reshape(1, D1), Wab, bab,
        Wc, Wcls, bcls.reshape(1, C))
    return (logits, prob, yhat)


# speedup vs baseline: 1.4985x; 1.0372x over previous
import functools

import jax
import jax.numpy as jnp
from jax.experimental import pallas as pl
from jax.experimental.pallas import tpu as pltpu

N, L, D1, D2, C = 50000, 1024, 256, 128, 2
TILE = 5000
NUM_TILES = N // TILE


def _clam_kernel(h_ref, w1_ref, b1_ref, wab_ref, bab_ref,
                 wc_ref, wcls_ref, bcls_ref,
                 logits_ref, prob_ref, yhat_ref,
                 acc_ref, m_ref, s_ref):
    i = pl.program_id(0)

    @pl.when(i == 0)
    def _init():
        acc_ref[...] = jnp.zeros_like(acc_ref)
        m_ref[...] = jnp.full((1, 1), -jnp.inf, jnp.float32)
        s_ref[...] = jnp.zeros((1, 1), jnp.float32)

    hh = jax.nn.relu(
        jnp.dot(h_ref[...].astype(jnp.bfloat16), w1_ref[...],
                preferred_element_type=jnp.float32)
        + b1_ref[...])
    ag = jnp.dot(hh.astype(jnp.bfloat16), wab_ref[...],
                 preferred_element_type=jnp.float32) + bab_ref[...]
    a = jnp.tanh(ag[:, :D2])
    g = jax.nn.sigmoid(ag[:, D2:])
    sc = jnp.dot(a * g, wc_ref[...],
                 preferred_element_type=jnp.float32)

    m_old = m_ref[...]
    m_new = jnp.maximum(m_old, jnp.max(sc, axis=0, keepdims=True))
    alpha = jnp.exp(m_old - m_new)
    p = jnp.exp(sc - m_new)
    s_ref[...] = s_ref[...] * alpha + jnp.sum(p, axis=0, keepdims=True)
    acc_ref[...] = acc_ref[...] * alpha + jnp.dot(
        p.T, hh, preferred_element_type=jnp.float32)
    m_ref[...] = m_new

    @pl.when(i == NUM_TILES - 1)
    def _finish():
        m_vec = acc_ref[...] / s_ref[...]
        logits = jnp.dot(m_vec, wcls_ref[...],
                         preferred_element_type=jnp.float32) + bcls_ref[...]
        logits_ref[...] = logits
        z = logits - jnp.max(logits, axis=1, keepdims=True)
        ez = jnp.exp(z)
        prob_ref[...] = ez / jnp.sum(ez, axis=1, keepdims=True)
        yhat_ref[...] = (logits[:, 1:2] > logits[:, 0:1]).astype(jnp.int32)


@jax.jit
def _run(h, W1, b1, Wab, bab, Wc, Wcls, bcls):
    out_shapes = (
        jax.ShapeDtypeStruct((1, C), jnp.float32),
        jax.ShapeDtypeStruct((1, C), jnp.float32),
        jax.ShapeDtypeStruct((1, 1), jnp.int32),
    )
    full = lambda shape: pl.BlockSpec(shape, lambda i: (0, 0))
    return pl.pallas_call(
        _clam_kernel,
        grid=(NUM_TILES,),
        in_specs=[
            pl.BlockSpec((TILE, L), lambda i: (i, 0)),
            full((L, D1)),
            full((1, D1)),
            full((D1, 2 * D2)),
            full((1, 2 * D2)),
            full((D2, 1)),
            full((D1, C)),
            full((1, C)),
        ],
        out_specs=(full((1, C)), full((1, C)), full((1, 1))),
        out_shape=out_shapes,
        scratch_shapes=[
            pltpu.VMEM((1, D1), jnp.float32),
            pltpu.VMEM((1, 1), jnp.float32),
            pltpu.VMEM((1, 1), jnp.float32),
        ],
    )(h, W1, b1, Wab, bab, Wc, Wcls, bcls)


def kernel(h, label, W1, b1, Wa, ba, Wb, bb, Wc, bc, Wcls, bcls):
    del label, bc
    Wab = jnp.concatenate([Wa, Wb], axis=1).astype(jnp.bfloat16)
    bab = jnp.concatenate([ba, bb]).reshape(1, 2 * D2)
    logits, prob, yhat = _run(
        h, W1.astype(jnp.bfloat16), b1.reshape(1, D1), Wab, bab,
        Wc, Wcls, bcls.reshape(1, C))
    return (logits, prob, yhat)
